# baseline (speedup 1.0000x reference)
import functools

import jax
import jax.numpy as jnp
from jax import lax
from jax.experimental import pallas as pl
from jax.experimental.pallas import tpu as pltpu
from jax.experimental.pallas import tpu_sc as plsc

NC = 2
NS = 16
NW = NC * NS
DP = 128
CHUNK = 128
NBUF = 3


@functools.lru_cache(maxsize=None)
def _emb_kernel(B):
    b_per_w = B // NW
    n_chunks = b_per_w // CHUNK

    mesh = plsc.VectorSubcoreMesh(
        core_axis_name="c", subcore_axis_name="s",
        num_cores=NC, num_subcores=NS)

    @functools.partial(
        pl.kernel,
        out_type=jax.ShapeDtypeStruct((B, DP), jnp.float32),
        mesh=mesh,
        scratch_types=[
            pltpu.VMEM((n_chunks, CHUNK), jnp.int32),
            pltpu.VMEM((NBUF, CHUNK, DP), jnp.float32),
            pltpu.SemaphoreType.DMA,
            pltpu.SemaphoreType.DMA,
        ],
        compiler_params=pltpu.CompilerParams(use_tc_tiling_on_sc=False),
    )
    def k(tok_hbm, table_hbm, out_hbm, idx_v, rows_v, gsem, wsem):
        wid = lax.axis_index("s") * NC + lax.axis_index("c")
        base = wid * b_per_w
        pltpu.sync_copy(tok_hbm.at[wid], idx_v)

        def fire_gather(s, b):
            pltpu.async_copy(table_hbm.at[idx_v.at[s]], rows_v.at[b], gsem)

        def wait_gather(s, b):
            pltpu.make_async_copy(
                table_hbm.at[idx_v.at[s]], rows_v.at[b], gsem).wait()

        def fire_write(s, b):
            pltpu.async_copy(
                rows_v.at[b], out_hbm.at[pl.ds(base + s * CHUNK, CHUNK)], wsem)

        def wait_write(s, b):
            pltpu.make_async_copy(
                rows_v.at[b], out_hbm.at[pl.ds(base + s * CHUNK, CHUNK)], wsem).wait()

        fire_gather(0, 0)
        fire_gather(1, 1)

        def body(s, carry):
            b = lax.rem(s, NBUF)
            wait_gather(s, b)
            fire_write(s, b)

            @pl.when(s >= 1)
            def _():
                wait_write(s - 1, lax.rem(s + 2, NBUF))

            @pl.when(s + 2 < n_chunks)
            def _():
                fire_gather(s + 2, lax.rem(s + 2, NBUF))

            return carry

        lax.fori_loop(0, n_chunks, body, 0)
        wait_write(n_chunks - 1, (n_chunks - 1) % NBUF)

    return k


def kernel(token_ids, weight):
    nb, nt = token_ids.shape
    B = nb * nt
    tok = token_ids.astype(jnp.int32).reshape(NW, B // (NW * CHUNK), CHUNK)
    wpad = jnp.pad(weight, ((0, 0), (0, DP - weight.shape[1])))
    out = _emb_kernel(B)(tok, wpad)
    return out[:, :weight.shape[1]].reshape(nb, nt, weight.shape[1])


# speedup vs baseline: 1.0023x; 1.0023x over previous
import functools

import jax
import jax.numpy as jnp
from jax import lax
from jax.experimental import pallas as pl
from jax.experimental.pallas import tpu as pltpu
from jax.experimental.pallas import tpu_sc as plsc

NC = 2
NS = 16
NW = NC * NS
DP = 128
CHUNK = 128
NBUF = 6
LOOKAHEAD = 4


@functools.lru_cache(maxsize=None)
def _emb_kernel(B):
    b_per_w = B // NW
    n_chunks = b_per_w // CHUNK

    mesh = plsc.VectorSubcoreMesh(
        core_axis_name="c", subcore_axis_name="s",
        num_cores=NC, num_subcores=NS)

    @functools.partial(
        pl.kernel,
        out_type=jax.ShapeDtypeStruct((B, DP), jnp.float32),
        mesh=mesh,
        scratch_types=[
            pltpu.VMEM((n_chunks, CHUNK), jnp.int32),
            pltpu.VMEM((NBUF, CHUNK, DP), jnp.float32),
            pltpu.SemaphoreType.DMA,
            pltpu.SemaphoreType.DMA,
        ],
        compiler_params=pltpu.CompilerParams(use_tc_tiling_on_sc=False),
    )
    def k(tok_hbm, table_hbm, out_hbm, idx_v, rows_v, gsem, wsem):
        wid = lax.axis_index("s") * NC + lax.axis_index("c")
        base = wid * b_per_w
        pltpu.sync_copy(tok_hbm.at[wid], idx_v)

        def fire_gather(s, b):
            pltpu.async_copy(table_hbm.at[idx_v.at[s]], rows_v.at[b], gsem)

        def wait_gather(s, b):
            pltpu.make_async_copy(
                table_hbm.at[idx_v.at[s]], rows_v.at[b], gsem).wait()

        def fire_write(s, b):
            pltpu.async_copy(
                rows_v.at[b], out_hbm.at[pl.ds(base + s * CHUNK, CHUNK)], wsem)

        def wait_write(s, b):
            pltpu.make_async_copy(
                rows_v.at[b], out_hbm.at[pl.ds(base + s * CHUNK, CHUNK)], wsem).wait()

        for s0 in range(LOOKAHEAD):
            fire_gather(s0, s0)

        def body(s, carry):
            b = lax.rem(s, NBUF)
            wait_gather(s, b)
            fire_write(s, b)

            @pl.when(s >= NBUF - LOOKAHEAD)
            def _():
                wait_write(s - (NBUF - LOOKAHEAD), lax.rem(s + LOOKAHEAD, NBUF))

            @pl.when(s + LOOKAHEAD < n_chunks)
            def _():
                fire_gather(s + LOOKAHEAD, lax.rem(s + LOOKAHEAD, NBUF))

            return carry

        lax.fori_loop(0, n_chunks, body, 0)
        for s0 in range(n_chunks - (NBUF - LOOKAHEAD), n_chunks):
            wait_write(s0, s0 % NBUF)

    return k


def kernel(token_ids, weight):
    nb, nt = token_ids.shape
    B = nb * nt
    tok = token_ids.astype(jnp.int32).reshape(NW, B // (NW * CHUNK), CHUNK)
    wpad = jnp.pad(weight, ((0, 0), (0, DP - weight.shape[1])))
    out = _emb_kernel(B)(tok, wpad)
    return out[:, :weight.shape[1]].reshape(nb, nt, weight.shape[1])


# speedup vs baseline: 1.0633x; 1.0609x over previous
import functools

import jax
import jax.numpy as jnp
from jax import lax
from jax.experimental import pallas as pl
from jax.experimental.pallas import tpu as pltpu
from jax.experimental.pallas import tpu_sc as plsc

NC = 2
NS = 16
NW = NC * NS
DP = 128
CHUNK = 128
NBUF = 6
LOOKAHEAD = 4


@functools.lru_cache(maxsize=None)
def _emb_kernel(B):
    b_per_w = B // NW
    n_chunks = b_per_w // CHUNK

    mesh = plsc.VectorSubcoreMesh(
        core_axis_name="c", subcore_axis_name="s",
        num_cores=NC, num_subcores=NS)

    @functools.partial(
        pl.kernel,
        out_type=jax.ShapeDtypeStruct((B, DP), jnp.float32),
        mesh=mesh,
        scratch_types=[
            pltpu.VMEM((n_chunks, CHUNK), jnp.int32),
            pltpu.VMEM((NBUF, CHUNK, DP), jnp.float32),
            pltpu.SemaphoreType.DMA,
            pltpu.SemaphoreType.DMA,
        ],
        compiler_params=pltpu.CompilerParams(use_tc_tiling_on_sc=False),
    )
    def k(tok_hbm, table_hbm, out_hbm, idx_v, rows_v, gsem, wsem):
        wid = lax.axis_index("s") * NC + lax.axis_index("c")
        base = wid * b_per_w
        pltpu.sync_copy(tok_hbm.at[wid], idx_v)

        def fire_gather(s, b):
            pltpu.async_copy(table_hbm.at[idx_v.at[s]], rows_v.at[b], gsem)

        def wait_gather(s, b):
            pltpu.make_async_copy(
                table_hbm.at[idx_v.at[s]], rows_v.at[b], gsem).wait()

        def fire_write(s, b):
            pltpu.async_copy(
                rows_v.at[b], out_hbm.at[pl.ds(base + s * CHUNK, CHUNK)], wsem)

        def wait_write(s, b):
            pltpu.make_async_copy(
                rows_v.at[b], out_hbm.at[pl.ds(base + s * CHUNK, CHUNK)], wsem).wait()

        for s0 in range(LOOKAHEAD):
            fire_gather(s0, s0)

        def body(s, carry):
            b = lax.rem(s, NBUF)
            wait_gather(s, b)
            fire_write(s, b)

            @pl.when(s >= NBUF - LOOKAHEAD)
            def _():
                wait_write(s - (NBUF - LOOKAHEAD), lax.rem(s + LOOKAHEAD, NBUF))

            @pl.when(s + LOOKAHEAD < n_chunks)
            def _():
                fire_gather(s + LOOKAHEAD, lax.rem(s + LOOKAHEAD, NBUF))

            return carry

        lax.fori_loop(0, n_chunks, body, 0)
        for s0 in range(n_chunks - (NBUF - LOOKAHEAD), n_chunks):
            wait_write(s0, s0 % NBUF)

    return k


TBLK = 2048


@functools.lru_cache(maxsize=None)
def _transpose_pad_kernel(V, d):
    n_blk = -(-V // TBLK)

    def tp(wt_ref, out_ref):
        x = wt_ref[...]
        y = jnp.transpose(x, (1, 0))
        out_ref[...] = jnp.pad(y, ((0, 0), (0, DP - d)))

    return pl.pallas_call(
        tp,
        grid=(n_blk,),
        in_specs=[pl.BlockSpec((d, TBLK), lambda j: (0, j))],
        out_specs=pl.BlockSpec((TBLK, DP), lambda j: (j, 0)),
        out_shape=jax.ShapeDtypeStruct((V, DP), jnp.float32),
    )


def kernel(token_ids, weight):
    nb, nt = token_ids.shape
    B = nb * nt
    tok = token_ids.astype(jnp.int32).reshape(NW, B // (NW * CHUNK), CHUNK)
    wpad = _transpose_pad_kernel(*weight.shape)(weight.T)
    out = _emb_kernel(B)(tok, wpad)
    return out[:, :weight.shape[1]].reshape(nb, nt, weight.shape[1])


# speedup vs baseline: 1.3482x; 1.2679x over previous
import functools

import jax
import jax.numpy as jnp
from jax import lax
from jax.experimental import pallas as pl
from jax.experimental.pallas import tpu as pltpu
from jax.experimental.pallas import tpu_sc as plsc

NC = 2
NS = 16
NW = NC * NS
DP = 128
CHUNK = 128
NBUF = 6
LOOKAHEAD = 4


@functools.lru_cache(maxsize=None)
def _emb_kernel(B):
    b_per_w = B // NW
    n_chunks = b_per_w // CHUNK

    mesh = plsc.VectorSubcoreMesh(
        core_axis_name="c", subcore_axis_name="s",
        num_cores=NC, num_subcores=NS)

    @functools.partial(
        pl.kernel,
        out_type=jax.ShapeDtypeStruct((B, DP), jnp.float32),
        mesh=mesh,
        scratch_types=[
            pltpu.VMEM((n_chunks, CHUNK), jnp.int32),
            pltpu.VMEM((NBUF, CHUNK, DP), jnp.float32),
            pltpu.SemaphoreType.DMA,
            pltpu.SemaphoreType.DMA,
        ],
        compiler_params=pltpu.CompilerParams(use_tc_tiling_on_sc=False),
    )
    def k(tok_hbm, table_hbm, out_hbm, idx_v, rows_v, gsem, wsem):
        wid = lax.axis_index("s") * NC + lax.axis_index("c")
        base = wid * b_per_w
        pltpu.sync_copy(tok_hbm.at[wid], idx_v)

        def fire_gather(s, b):
            pltpu.async_copy(table_hbm.at[idx_v.at[s]], rows_v.at[b], gsem)

        def wait_gather(s, b):
            pltpu.make_async_copy(
                table_hbm.at[idx_v.at[s]], rows_v.at[b], gsem).wait()

        def fire_write(s, b):
            pltpu.async_copy(
                rows_v.at[b], out_hbm.at[pl.ds(base + s * CHUNK, CHUNK)], wsem)

        def wait_write(s, b):
            pltpu.make_async_copy(
                rows_v.at[b], out_hbm.at[pl.ds(base + s * CHUNK, CHUNK)], wsem).wait()

        for s0 in range(LOOKAHEAD):
            fire_gather(s0, s0)

        def body(s, carry):
            b = lax.rem(s, NBUF)
            wait_gather(s, b)
            fire_write(s, b)

            @pl.when(s >= NBUF - LOOKAHEAD)
            def _():
                wait_write(s - (NBUF - LOOKAHEAD), lax.rem(s + LOOKAHEAD, NBUF))

            @pl.when(s + LOOKAHEAD < n_chunks)
            def _():
                fire_gather(s + LOOKAHEAD, lax.rem(s + LOOKAHEAD, NBUF))

            return carry

        lax.fori_loop(0, n_chunks, body, 0)
        for s0 in range(n_chunks - (NBUF - LOOKAHEAD), n_chunks):
            wait_write(s0, s0 % NBUF)

    return k


TBLK = 8192


@functools.lru_cache(maxsize=None)
def _transpose_pad_kernel(V, d):
    n_blk = -(-V // TBLK)

    def tp(wt_ref, out_ref):
        x = wt_ref[...]
        y = jnp.transpose(x, (1, 0))
        out_ref[...] = jnp.pad(y, ((0, 0), (0, DP - d)))

    return pl.pallas_call(
        tp,
        grid=(n_blk,),
        in_specs=[pl.BlockSpec((d, TBLK), lambda j: (0, j))],
        out_specs=pl.BlockSpec((TBLK, DP), lambda j: (j, 0)),
        out_shape=jax.ShapeDtypeStruct((V, DP), jnp.float32),
    )


def kernel(token_ids, weight):
    nb, nt = token_ids.shape
    B = nb * nt
    tok = token_ids.astype(jnp.int32).reshape(NW, B // (NW * CHUNK), CHUNK)
    wpad = _transpose_pad_kernel(*weight.shape)(weight.T)
    out = _emb_kernel(B)(tok, wpad)
    return out[:, :weight.shape[1]].reshape(nb, nt, weight.shape[1])


# speedup vs baseline: 1.3796x; 1.0233x over previous
import functools

import jax
import jax.numpy as jnp
from jax import lax
from jax.experimental import pallas as pl
from jax.experimental.pallas import tpu as pltpu
from jax.experimental.pallas import tpu_sc as plsc

NC = 2
NS = 16
NW = NC * NS
DP = 128
CHUNK = 128
NBUF = 6
LOOKAHEAD = 4


@functools.lru_cache(maxsize=None)
def _emb_kernel(B):
    b_per_w = B // NW
    n_chunks = b_per_w // CHUNK

    mesh = plsc.VectorSubcoreMesh(
        core_axis_name="c", subcore_axis_name="s",
        num_cores=NC, num_subcores=NS)

    @functools.partial(
        pl.kernel,
        out_type=jax.ShapeDtypeStruct((B, DP), jnp.float32),
        mesh=mesh,
        scratch_types=[
            pltpu.VMEM((n_chunks, CHUNK), jnp.int32),
            pltpu.VMEM((NBUF, CHUNK, DP), jnp.float32),
            pltpu.SemaphoreType.DMA,
            pltpu.SemaphoreType.DMA,
        ],
        compiler_params=pltpu.CompilerParams(use_tc_tiling_on_sc=False),
    )
    def k(tok_hbm, table_hbm, out_hbm, idx_v, rows_v, gsem, wsem):
        wid = lax.axis_index("s") * NC + lax.axis_index("c")
        base = wid * b_per_w
        pltpu.sync_copy(tok_hbm.at[wid], idx_v)

        def fire_gather(s, b):
            pltpu.async_copy(table_hbm.at[idx_v.at[s]], rows_v.at[b], gsem)

        def wait_gather(s, b):
            pltpu.make_async_copy(
                table_hbm.at[idx_v.at[s]], rows_v.at[b], gsem).wait()

        def fire_write(s, b):
            pltpu.async_copy(
                rows_v.at[b], out_hbm.at[pl.ds(base + s * CHUNK, CHUNK)], wsem)

        def wait_write(s, b):
            pltpu.make_async_copy(
                rows_v.at[b], out_hbm.at[pl.ds(base + s * CHUNK, CHUNK)], wsem).wait()

        for s0 in range(LOOKAHEAD):
            fire_gather(s0, s0)

        def body(s, carry):
            b = lax.rem(s, NBUF)
            wait_gather(s, b)
            fire_write(s, b)

            @pl.when(s >= NBUF - LOOKAHEAD)
            def _():
                wait_write(s - (NBUF - LOOKAHEAD), lax.rem(s + LOOKAHEAD, NBUF))

            @pl.when(s + LOOKAHEAD < n_chunks)
            def _():
                fire_gather(s + LOOKAHEAD, lax.rem(s + LOOKAHEAD, NBUF))

            return carry

        lax.fori_loop(0, n_chunks, body, 0)
        for s0 in range(n_chunks - (NBUF - LOOKAHEAD), n_chunks):
            wait_write(s0, s0 % NBUF)

    return k


TBLK = 16384


@functools.lru_cache(maxsize=None)
def _transpose_pad_kernel(V, d):
    n_blk = -(-V // TBLK)

    def tp(wt_ref, out_ref):
        x = wt_ref[...]
        y = jnp.transpose(x, (1, 0))
        out_ref[...] = jnp.pad(y, ((0, 0), (0, DP - d)))

    return pl.pallas_call(
        tp,
        grid=(n_blk,),
        in_specs=[pl.BlockSpec((d, TBLK), lambda j: (0, j))],
        out_specs=pl.BlockSpec((TBLK, DP), lambda j: (j, 0)),
        out_shape=jax.ShapeDtypeStruct((V, DP), jnp.float32),
    )


def kernel(token_ids, weight):
    nb, nt = token_ids.shape
    B = nb * nt
    tok = token_ids.astype(jnp.int32).reshape(NW, B // (NW * CHUNK), CHUNK)
    wpad = _transpose_pad_kernel(*weight.shape)(weight.T)
    out = _emb_kernel(B)(tok, wpad)
    return out[:, :weight.shape[1]].reshape(nb, nt, weight.shape[1])


# speedup vs baseline: 1.3919x; 1.0089x over previous
import functools

import jax
import jax.numpy as jnp
from jax import lax
from jax.experimental import pallas as pl
from jax.experimental.pallas import tpu as pltpu
from jax.experimental.pallas import tpu_sc as plsc

NC = 2
NS = 16
NW = NC * NS
DP = 128
CHUNK = 128
NBUF = 6
LOOKAHEAD = 4


@functools.lru_cache(maxsize=None)
def _emb_kernel(B):
    b_per_w = B // NW
    n_chunks = b_per_w // CHUNK

    mesh = plsc.VectorSubcoreMesh(
        core_axis_name="c", subcore_axis_name="s",
        num_cores=NC, num_subcores=NS)

    @functools.partial(
        pl.kernel,
        out_type=jax.ShapeDtypeStruct((B, DP), jnp.float32),
        mesh=mesh,
        scratch_types=[
            pltpu.VMEM((n_chunks, CHUNK), jnp.int32),
            pltpu.VMEM((NBUF, CHUNK, DP), jnp.float32),
            pltpu.SemaphoreType.DMA,
            pltpu.SemaphoreType.DMA,
        ],
        compiler_params=pltpu.CompilerParams(use_tc_tiling_on_sc=False),
    )
    def k(tok_hbm, table_hbm, out_hbm, idx_v, rows_v, gsem, wsem):
        wid = lax.axis_index("s") * NC + lax.axis_index("c")
        base = wid * b_per_w
        pltpu.sync_copy(tok_hbm.at[wid], idx_v)

        def fire_gather(s, b):
            pltpu.async_copy(table_hbm.at[idx_v.at[s]], rows_v.at[b], gsem)

        def wait_gather(s, b):
            pltpu.make_async_copy(
                table_hbm.at[idx_v.at[s]], rows_v.at[b], gsem).wait()

        def fire_write(s, b):
            pltpu.async_copy(
                rows_v.at[b], out_hbm.at[pl.ds(base + s * CHUNK, CHUNK)], wsem)

        def wait_write(s, b):
            pltpu.make_async_copy(
                rows_v.at[b], out_hbm.at[pl.ds(base + s * CHUNK, CHUNK)], wsem).wait()

        for s0 in range(LOOKAHEAD):
            fire_gather(s0, s0)

        def body(s, carry):
            b = lax.rem(s, NBUF)
            wait_gather(s, b)
            fire_write(s, b)

            @pl.when(s >= NBUF - LOOKAHEAD)
            def _():
                wait_write(s - (NBUF - LOOKAHEAD), lax.rem(s + LOOKAHEAD, NBUF))

            @pl.when(s + LOOKAHEAD < n_chunks)
            def _():
                fire_gather(s + LOOKAHEAD, lax.rem(s + LOOKAHEAD, NBUF))

            return carry

        lax.fori_loop(0, n_chunks, body, 0)
        for s0 in range(n_chunks - (NBUF - LOOKAHEAD), n_chunks):
            wait_write(s0, s0 % NBUF)

    return k


TBLK = 32768


@functools.lru_cache(maxsize=None)
def _transpose_pad_kernel(V, d):
    n_blk = -(-V // TBLK)

    def tp(wt_ref, out_ref):
        x = wt_ref[...]
        y = jnp.transpose(x, (1, 0))
        out_ref[...] = jnp.pad(y, ((0, 0), (0, DP - d)))

    return pl.pallas_call(
        tp,
        grid=(n_blk,),
        in_specs=[pl.BlockSpec((d, TBLK), lambda j: (0, j))],
        out_specs=pl.BlockSpec((TBLK, DP), lambda j: (j, 0)),
        out_shape=jax.ShapeDtypeStruct((V, DP), jnp.float32),
    )


def kernel(token_ids, weight):
    nb, nt = token_ids.shape
    B = nb * nt
    tok = token_ids.astype(jnp.int32).reshape(NW, B // (NW * CHUNK), CHUNK)
    wpad = _transpose_pad_kernel(*weight.shape)(weight.T)
    out = _emb_kernel(B)(tok, wpad)
    return out[:, :weight.shape[1]].reshape(nb, nt, weight.shape[1])
